# Initial kernel scaffold; baseline (speedup 1.0000x reference)
import jax
import jax.numpy as jnp
from jax.experimental import pallas as pl


def kernel(h, cell, x, edge_index, Wl_f, bl_f, Wr_f, Wg_f, bg_f, Wl_i, bl_i, Wr_i, Wg_i, bg_i, Wl_c, bl_c, Wr_c, Wg_c, bg_c, Wl_o, bl_o, Wr_o, Wg_o, bg_o, ln_g, ln_b):
    raise NotImplementedError("write your pallas kernel here")


# speedup vs baseline: 5.9398x; 5.9398x over previous
import dataclasses
import functools

import jax
import jax.numpy as jnp
from jax import lax
from jax.experimental import pallas as pl
from jax.experimental.pallas import tpu as pltpu
from jax.experimental.pallas import tpu_sc as plsc

N = 10000
E = 320000
D = 128
NC = 2
NS = 16
NW = NC * NS
EPW = E // NW
CHUNK = 80
NCH = EPW // CHUNK
SR = 80
NSTR = N // SR
SPS = (NSTR + NS - 1) // NS
L = 16

_mesh = plsc.VectorSubcoreMesh(core_axis_name="c", subcore_axis_name="s")

_sc_params = pltpu.CompilerParams()
if "needs_layout_passes" in pltpu.CompilerParams.__dataclass_fields__:
    _sc_params = dataclasses.replace(_sc_params, needs_layout_passes=False)


def _sc_segsum(h, edges_rs, zeros_blk):
    @functools.partial(
        pl.kernel,
        out_type=(
            jax.ShapeDtypeStruct((NC, N, D), jnp.float32),
            jax.ShapeDtypeStruct((NW, N), jnp.float32),
        ),
        mesh=_mesh,
        scratch_types=[
            pltpu.VMEM((1, CHUNK), jnp.int32),
            pltpu.VMEM((1, CHUNK), jnp.int32),
            pltpu.VMEM((CHUNK, D), jnp.float32),
            pltpu.VMEM((N,), jnp.float32),
            pltpu.VMEM_SHARED((N, D), jnp.float32),
            pltpu.SemaphoreType.DMA,
        ],
        compiler_params=_sc_params,
    )
    def k(h_hbm, edges_hbm, zeros_hbm, rows_out, cnt_out, src_v, dst_v,
          rows_v, cnt_v, acc_sh, sem):
        cid = lax.axis_index("c")
        sid = lax.axis_index("s")
        wid = cid * NS + sid

        @pl.loop(0, N // L)
        def _(i):
            cnt_v[pl.ds(i * L, L)] = jnp.zeros((L,), jnp.float32)

        @pl.loop(0, SPS)
        def _(b):
            c = sid + b * NS

            @pl.when(c < NSTR)
            def _():
                pltpu.sync_copy(zeros_hbm, acc_sh.at[pl.ds(c * SR, SR)])

        plsc.subcore_barrier()

        @pl.loop(0, NCH)
        def _(t):
            pltpu.sync_copy(edges_hbm.at[0, wid, t], src_v)
            pltpu.sync_copy(edges_hbm.at[1, wid, t], dst_v)
            gather = pltpu.async_copy(h_hbm.at[src_v.at[0]], rows_v, sem)

            for kk in range(CHUNK // L):
                idx = dst_v[0, pl.ds(kk * L, L)]
                run, last = plsc.scan_count(idx)
                cur = plsc.load_gather(cnt_v, [idx])
                plsc.store_scatter(cnt_v, [idx],
                                   cur + run.astype(jnp.float32), mask=last)

            gather.wait()
            pltpu.sync_copy(rows_v, acc_sh.at[dst_v.at[0]], add=True)

        plsc.subcore_barrier()

        pltpu.sync_copy(cnt_v, cnt_out.at[wid])

        @pl.loop(0, SPS)
        def _(b):
            c = sid + b * NS

            @pl.when(c < NSTR)
            def _():
                pltpu.sync_copy(
                    acc_sh.at[pl.ds(c * SR, SR)],
                    rows_out.at[cid, pl.ds(c * SR, SR)],
                )

    return k(h, edges_rs, zeros_blk)


def _tc_body(h_ref, x_ref, cell_ref, acc_ref, cnt_ref, wh_ref, wx_ref, wm_ref,
             b_ref, g_ref, be_ref, ones_ref, hn_ref, cn_ref):
    s = acc_ref[0] + acc_ref[1]
    cnt = jnp.dot(cnt_ref[...], ones_ref[...],
                  preferred_element_type=jnp.float32)
    mean = s / jnp.maximum(cnt, 1.0)
    pre = (
        jnp.dot(h_ref[...], wh_ref[...], preferred_element_type=jnp.float32)
        + jnp.dot(x_ref[...], wx_ref[...], preferred_element_type=jnp.float32)
        + jnp.dot(mean, wm_ref[...], preferred_element_type=jnp.float32)
        + b_ref[...]
    )
    f = jax.nn.sigmoid(pre[:, 0 * D:1 * D])
    i = jax.nn.sigmoid(pre[:, 1 * D:2 * D])
    ct = jnp.tanh(pre[:, 2 * D:3 * D])
    o = jax.nn.sigmoid(pre[:, 3 * D:4 * D])
    cn = f * cell_ref[...] + i * ct
    hn = o * jnp.tanh(cn)
    mu = jnp.mean(hn, axis=1, keepdims=True)
    dlt = hn - mu
    var = jnp.mean(dlt * dlt, axis=1, keepdims=True)
    hn_ref[...] = dlt * lax.rsqrt(var + 1e-5) * g_ref[...] + be_ref[...]
    cn_ref[...] = cn


def _tc_fused(h, x, cell, acc, cnt, Wh, Wx, Wm, bias, ln_g, ln_b):
    R = 1000
    row_spec = pl.BlockSpec((R, D), lambda i: (i, 0))
    full = lambda shape: pl.BlockSpec(shape, lambda i: tuple(0 for _ in shape))
    return pl.pallas_call(
        _tc_body,
        grid=(N // R,),
        in_specs=[
            row_spec, row_spec, row_spec,
            pl.BlockSpec((NC, R, D), lambda i: (0, i, 0)),
            pl.BlockSpec((R, NW), lambda i: (i, 0)),
            full((D, 4 * D)), full((D, 4 * D)), full((D, 4 * D)),
            full((1, 4 * D)), full((1, D)), full((1, D)),
            full((NW, 1)),
        ],
        out_specs=[row_spec, row_spec],
        out_shape=[
            jax.ShapeDtypeStruct((N, D), jnp.float32),
            jax.ShapeDtypeStruct((N, D), jnp.float32),
        ],
    )(h, x, cell, acc, cnt, Wh, Wx, Wm, bias, ln_g, ln_b,
      jnp.ones((NW, 1), jnp.float32))


def kernel(h, cell, x, edge_index, Wl_f, bl_f, Wr_f, Wg_f, bg_f, Wl_i, bl_i,
           Wr_i, Wg_i, bg_i, Wl_c, bl_c, Wr_c, Wg_c, bg_c, Wl_o, bl_o, Wr_o,
           Wg_o, bg_o, ln_g, ln_b):
    Wgs = [Wg_f, Wg_i, Wg_c, Wg_o]
    Wrs = [Wr_f, Wr_i, Wr_c, Wr_o]
    Wls = [Wl_f, Wl_i, Wl_c, Wl_o]
    bgs = [bg_f, bg_i, bg_c, bg_o]
    bls = [bl_f, bl_i, bl_c, bl_o]
    Wh = jnp.concatenate([(Wg[:, :D] + Wr).T for Wg, Wr in zip(Wgs, Wrs)], axis=1)
    Wx = jnp.concatenate([Wg[:, D:].T for Wg in Wgs], axis=1)
    Wm = jnp.concatenate([Wl.T for Wl in Wls], axis=1)
    bias = jnp.concatenate([bg + bl for bg, bl in zip(bgs, bls)])[None, :]

    edges_rs = edge_index.reshape(2, NW, NCH, 1, CHUNK)

    acc, cnt = _sc_segsum(h, edges_rs, jnp.zeros((SR, D), jnp.float32))
    h_new, cell_new = _tc_fused(
        h, x, cell, acc, cnt.T, Wh, Wx, Wm, bias, ln_g[None, :], ln_b[None, :]
    )
    return (h_new, cell_new)


# speedup vs baseline: 12.4017x; 2.0879x over previous
import dataclasses
import functools

import jax
import jax.numpy as jnp
from jax import lax
from jax.experimental import pallas as pl
from jax.experimental.pallas import tpu as pltpu
from jax.experimental.pallas import tpu_sc as plsc

N = 10000
E = 320000
D = 128
NC = 2
NS = 16
NW = NC * NS
EPW = E // NW
CHUNK = 80
NCH = EPW // CHUNK
SR = 80
NSTR = N // SR
SPS = (NSTR + NS - 1) // NS
L = 16

_mesh = plsc.VectorSubcoreMesh(core_axis_name="c", subcore_axis_name="s")

_sc_params = pltpu.CompilerParams()
if "needs_layout_passes" in pltpu.CompilerParams.__dataclass_fields__:
    _sc_params = dataclasses.replace(_sc_params, needs_layout_passes=False)


def _sc_segsum(h, edges_rs, zeros_blk):
    @functools.partial(
        pl.kernel,
        out_type=(
            jax.ShapeDtypeStruct((NC, N, D), jnp.float32),
            jax.ShapeDtypeStruct((NW, N), jnp.float32),
        ),
        mesh=_mesh,
        scratch_types=[
            pltpu.VMEM((EPW,), jnp.int32),
            pltpu.VMEM((EPW,), jnp.int32),
            pltpu.VMEM((CHUNK, D), jnp.float32),
            pltpu.VMEM((CHUNK, D), jnp.float32),
            pltpu.VMEM((N,), jnp.float32),
            pltpu.VMEM_SHARED((N, D), jnp.float32),
            pltpu.SemaphoreType.DMA,
            pltpu.SemaphoreType.DMA,
        ],
        compiler_params=_sc_params,
    )
    def k(h_hbm, edges_hbm, zeros_hbm, rows_out, cnt_out, src_v, dst_v,
          rows0_v, rows1_v, cnt_v, acc_sh, sem0, sem1):
        cid = lax.axis_index("c")
        sid = lax.axis_index("s")
        wid = cid * NS + sid

        pltpu.sync_copy(edges_hbm.at[0, wid], src_v)
        pltpu.sync_copy(edges_hbm.at[1, wid], dst_v)

        @pl.loop(0, N // L)
        def _(i):
            cnt_v[pl.ds(i * L, L)] = jnp.zeros((L,), jnp.float32)

        @pl.loop(0, SPS)
        def _(b):
            c = sid + b * NS

            @pl.when(c < NSTR)
            def _():
                pltpu.sync_copy(zeros_hbm, acc_sh.at[pl.ds(c * SR, SR)])

        plsc.subcore_barrier()

        def count(t):
            for kk in range(CHUNK // L):
                idx = dst_v[pl.ds(t * CHUNK + kk * L, L)]
                run, last = plsc.scan_count(idx)
                cur = plsc.load_gather(cnt_v, [idx])
                plsc.store_scatter(cnt_v, [idx],
                                   cur + run.astype(jnp.float32), mask=last)

        def src_at(t):
            return src_v.at[pl.ds(t * CHUNK, CHUNK)]

        def dst_at(t):
            return dst_v.at[pl.ds(t * CHUNK, CHUNK)]

        def wait0():
            pltpu.make_async_copy(h_hbm.at[src_at(0)], rows0_v, sem0).wait()

        def wait1():
            pltpu.make_async_copy(h_hbm.at[src_at(0)], rows1_v, sem1).wait()

        pltpu.async_copy(h_hbm.at[src_at(0)], rows0_v, sem0)

        @pl.loop(0, (NCH - 1) // 2)
        def _(i):
            t0 = 2 * i
            pltpu.async_copy(h_hbm.at[src_at(t0 + 1)], rows1_v, sem1)
            wait0()
            count(t0)
            pltpu.sync_copy(rows0_v, acc_sh.at[dst_at(t0)], add=True)
            pltpu.async_copy(h_hbm.at[src_at(t0 + 2)], rows0_v, sem0)
            wait1()
            count(t0 + 1)
            pltpu.sync_copy(rows1_v, acc_sh.at[dst_at(t0 + 1)], add=True)

        wait0()
        count(NCH - 1)
        pltpu.sync_copy(rows0_v, acc_sh.at[dst_at(NCH - 1)], add=True)

        plsc.subcore_barrier()

        pltpu.sync_copy(cnt_v, cnt_out.at[wid])

        @pl.loop(0, SPS)
        def _(b):
            c = sid + b * NS

            @pl.when(c < NSTR)
            def _():
                pltpu.sync_copy(
                    acc_sh.at[pl.ds(c * SR, SR)],
                    rows_out.at[cid, pl.ds(c * SR, SR)],
                )

    return k(h, edges_rs, zeros_blk)


def _tc_body(h_ref, x_ref, cell_ref, acc_ref, cnt_ref, wh_ref, wx_ref, wm_ref,
             b_ref, g_ref, be_ref, ones_ref, hn_ref, cn_ref):
    s = acc_ref[0] + acc_ref[1]
    cnt = jnp.dot(cnt_ref[...], ones_ref[...],
                  preferred_element_type=jnp.float32)
    mean = s / jnp.maximum(cnt, 1.0)
    pre = (
        jnp.dot(h_ref[...], wh_ref[...], preferred_element_type=jnp.float32)
        + jnp.dot(x_ref[...], wx_ref[...], preferred_element_type=jnp.float32)
        + jnp.dot(mean, wm_ref[...], preferred_element_type=jnp.float32)
        + b_ref[...]
    )
    f = jax.nn.sigmoid(pre[:, 0 * D:1 * D])
    i = jax.nn.sigmoid(pre[:, 1 * D:2 * D])
    ct = jnp.tanh(pre[:, 2 * D:3 * D])
    o = jax.nn.sigmoid(pre[:, 3 * D:4 * D])
    cn = f * cell_ref[...] + i * ct
    hn = o * jnp.tanh(cn)
    mu = jnp.mean(hn, axis=1, keepdims=True)
    dlt = hn - mu
    var = jnp.mean(dlt * dlt, axis=1, keepdims=True)
    hn_ref[...] = dlt * lax.rsqrt(var + 1e-5) * g_ref[...] + be_ref[...]
    cn_ref[...] = cn


def _tc_fused(h, x, cell, acc, cnt, Wh, Wx, Wm, bias, ln_g, ln_b):
    R = 1000
    row_spec = pl.BlockSpec((R, D), lambda i: (i, 0))
    full = lambda shape: pl.BlockSpec(shape, lambda i: tuple(0 for _ in shape))
    return pl.pallas_call(
        _tc_body,
        grid=(N // R,),
        in_specs=[
            row_spec, row_spec, row_spec,
            pl.BlockSpec((NC, R, D), lambda i: (0, i, 0)),
            pl.BlockSpec((R, NW), lambda i: (i, 0)),
            full((D, 4 * D)), full((D, 4 * D)), full((D, 4 * D)),
            full((1, 4 * D)), full((1, D)), full((1, D)),
            full((NW, 1)),
        ],
        out_specs=[row_spec, row_spec],
        out_shape=[
            jax.ShapeDtypeStruct((N, D), jnp.float32),
            jax.ShapeDtypeStruct((N, D), jnp.float32),
        ],
    )(h, x, cell, acc, cnt, Wh, Wx, Wm, bias, ln_g, ln_b,
      jnp.ones((NW, 1), jnp.float32))


def kernel(h, cell, x, edge_index, Wl_f, bl_f, Wr_f, Wg_f, bg_f, Wl_i, bl_i,
           Wr_i, Wg_i, bg_i, Wl_c, bl_c, Wr_c, Wg_c, bg_c, Wl_o, bl_o, Wr_o,
           Wg_o, bg_o, ln_g, ln_b):
    Wgs = [Wg_f, Wg_i, Wg_c, Wg_o]
    Wrs = [Wr_f, Wr_i, Wr_c, Wr_o]
    Wls = [Wl_f, Wl_i, Wl_c, Wl_o]
    bgs = [bg_f, bg_i, bg_c, bg_o]
    bls = [bl_f, bl_i, bl_c, bl_o]
    Wh = jnp.concatenate([(Wg[:, :D] + Wr).T for Wg, Wr in zip(Wgs, Wrs)], axis=1)
    Wx = jnp.concatenate([Wg[:, D:].T for Wg in Wgs], axis=1)
    Wm = jnp.concatenate([Wl.T for Wl in Wls], axis=1)
    bias = jnp.concatenate([bg + bl for bg, bl in zip(bgs, bls)])[None, :]

    edges_rs = edge_index.reshape(2, NW, EPW)

    acc, cnt = _sc_segsum(h, edges_rs, jnp.zeros((SR, D), jnp.float32))
    h_new, cell_new = _tc_fused(
        h, x, cell, acc, cnt.T, Wh, Wx, Wm, bias, ln_g[None, :], ln_b[None, :]
    )
    return (h_new, cell_new)
